# baseline (speedup 1.0000x reference)
import functools

import jax
import jax.numpy as jnp
from jax import lax
from jax.experimental import pallas as pl
from jax.experimental.pallas import tpu as pltpu
from jax.experimental.pallas import tpu_sc as plsc

NC, NS, L = 2, 16, 16
NW = NC * NS


def kernel(x_sparse, x_dense, table, W_dense):
    B, F = x_sparse.shape
    _, V = table.shape
    _, D = x_dense.shape

    b_per_w = B // NW
    n_chunks = b_per_w // L

    xs_flat = x_sparse.astype(jnp.int32).reshape(-1)
    xd_flat = x_dense.reshape(-1)
    tab_flat = table.reshape(-1)
    w_blk = jnp.broadcast_to(W_dense, (D, L))

    rows = jnp.arange(NW, dtype=jnp.int32)[:, None, None] * b_per_w + jnp.arange(
        b_per_w, dtype=jnp.int32
    )[None, None, :]
    pos_s = rows * F + jnp.arange(F, dtype=jnp.int32)[None, :, None]
    pos_d = rows * D + jnp.arange(D, dtype=jnp.int32)[None, :, None]

    mesh = plsc.VectorSubcoreMesh(
        core_axis_name="c", subcore_axis_name="s", num_cores=NC, num_subcores=NS
    )

    @functools.partial(
        pl.kernel,
        out_type=jax.ShapeDtypeStruct((B,), jnp.float32),
        mesh=mesh,
        scratch_types=[
            pltpu.VMEM((F, b_per_w), jnp.int32),
            pltpu.VMEM((D, b_per_w), jnp.int32),
            pltpu.VMEM((F, b_per_w), jnp.int32),
            pltpu.VMEM((F, b_per_w), jnp.float32),
            pltpu.VMEM((D, b_per_w), jnp.float32),
            pltpu.VMEM((D, L), jnp.float32),
            pltpu.VMEM((b_per_w,), jnp.float32),
            pltpu.SemaphoreType.DMA,
            pltpu.SemaphoreType.DMA,
        ],
    )
    def sc_kernel(
        xs_hbm, xd_hbm, tab_hbm, w_hbm, ps_hbm, pd_hbm, out_hbm,
        ps_v, pd_v, idx_v, vals_v, xd_v, w_v, acc_v, sem1, sem2,
    ):
        wid = lax.axis_index("s") * NC + lax.axis_index("c")
        base = wid * b_per_w

        pltpu.sync_copy(ps_hbm.at[wid], ps_v)
        pltpu.sync_copy(pd_hbm.at[wid], pd_v)
        pltpu.sync_copy(w_hbm, w_v)

        idx_copies = [
            pltpu.async_copy(xs_hbm.at[ps_v.at[f]], idx_v.at[f], sem1)
            for f in range(F)
        ]
        dense_copies = [
            pltpu.async_copy(xd_hbm.at[pd_v.at[d]], xd_v.at[d], sem2)
            for d in range(D)
        ]

        tab_copies = []
        for f in range(F):
            idx_copies[f].wait()
            for c in range(n_chunks):
                sl = pl.ds(c * L, L)
                idx_v[f, sl] = idx_v[f, sl] + f * V
            tab_copies.append(
                pltpu.async_copy(tab_hbm.at[idx_v.at[f]], vals_v.at[f], sem2)
            )

        for cp in dense_copies:
            cp.wait()
        for cp in tab_copies:
            cp.wait()

        w_bcast = [w_v[d, :] for d in range(D)]
        for c in range(n_chunks):
            sl = pl.ds(c * L, L)
            acc = vals_v[0, sl]
            for f in range(1, F):
                acc = acc + vals_v[f, sl]
            for d in range(D):
                acc = acc + xd_v[d, sl] * w_bcast[d]
            acc_v[sl] = acc

        pltpu.sync_copy(acc_v, out_hbm.at[pl.ds(base, b_per_w)])

    out = sc_kernel(xs_flat, xd_flat, tab_flat, w_blk, pos_s, pos_d)
    return out.reshape(B, 1)


# speedup vs baseline: 1.1102x; 1.1102x over previous
import functools

import jax
import jax.numpy as jnp
from jax import lax
from jax.experimental import pallas as pl
from jax.experimental.pallas import tpu as pltpu
from jax.experimental.pallas import tpu_sc as plsc

NC, NS, L = 2, 16, 16


def kernel(x_sparse, x_dense, table, W_dense):
    B, F = x_sparse.shape
    _, V = table.shape

    rows_pw = B // NS
    n_half = rows_pw // 128
    n_chunks_h = 128 // L

    v_main = V // 128 * 128
    n_tiles_main = v_main // 128
    band_w = 8 * v_main
    n_band_fields = F // 8 * 8
    n_tail_f = F - n_band_fields
    w_tail = V - v_main
    tail_cols_n = w_tail * F

    trb = band_w
    tcb = band_w + V
    sp_words = tcb + tail_cols_n

    per = [n_tiles_main // NS + (1 if i < n_tiles_main % NS else 0) for i in range(NS)]
    starts = [sum(per[:i]) for i in range(NS)]
    buf_w = max(per) * 128
    tr_per = (V // NS + 7) // 8 * 8
    tr_chunks = [(i * tr_per, max(0, min(tr_per, V - i * tr_per))) for i in range(NS)]

    xs_blk = (
        x_sparse.astype(jnp.int32)
        .T.reshape(F, NS, n_half, 128)
        .transpose(1, 0, 2, 3)
    )
    tail_rows = table[n_band_fields:, :].reshape(-1)
    tail_cols = table[:, v_main:].reshape(-1)

    mesh = plsc.VectorSubcoreMesh(
        core_axis_name="c", subcore_axis_name="s", num_cores=NC, num_subcores=NS
    )

    @functools.partial(
        pl.kernel,
        out_type=jax.ShapeDtypeStruct((NC, B), jnp.float32),
        mesh=mesh,
        scratch_types=[
            pltpu.VMEM_SHARED((sp_words,), jnp.float32),
            pltpu.VMEM((8, buf_w), jnp.float32),
            pltpu.VMEM((tr_per,), jnp.float32),
            pltpu.VMEM((tail_cols_n,), jnp.float32),
            pltpu.VMEM((F, n_half, 128), jnp.int32),
            pltpu.VMEM((F, n_half, 128), jnp.float32),
            pltpu.VMEM((rows_pw,), jnp.float32),
            pltpu.SemaphoreType.DMA,
            pltpu.SemaphoreType.DMA,
            pltpu.SemaphoreType.DMA,
        ],
    )
    def sc_kernel(
        xs_hbm, tr_hbm, tc_hbm, tab_hbm, out_hbm,
        sp, stage_v, tr_v, tc_v, idx_v, vals_v, acc_v, sem_st, sem_push, sem,
    ):
        c = lax.axis_index("c")
        s = lax.axis_index("s")

        pltpu.sync_copy(xs_hbm.at[s], idx_v)

        def stage_band(band, t0, w):
            pltpu.async_copy(
                tab_hbm.at[pl.ds(band * 8, 8), pl.ds(t0 * 128, w)],
                stage_v.at[:, pl.ds(0, w)],
                sem_st,
            ).wait()
            pushes = [
                pltpu.async_copy(
                    stage_v.at[r, pl.ds(0, w)],
                    sp.at[pl.ds(r * v_main + t0 * 128, w)],
                    sem_push,
                )
                for r in range(8)
            ]
            for cp in pushes:
                cp.wait()

        def stage_tails():
            pltpu.sync_copy(tc_hbm, tc_v)
            pltpu.sync_copy(tc_v, sp.at[pl.ds(tcb, tail_cols_n)])

        def stage_tail_row(row_base, si):
            o, ln = tr_chunks[si]
            if ln > 0:
                pltpu.sync_copy(tr_hbm.at[pl.ds(row_base + o, ln)], tr_v.at[pl.ds(0, ln)])
                pltpu.sync_copy(tr_v.at[pl.ds(0, ln)], sp.at[pl.ds(trb + o, ln)])

        def offsets_and_gather(fields, f_row):
            for f in fields:
                for h in range(n_half):
                    for ch in range(n_chunks_h):
                        sl = pl.ds(ch * L, L)
                        v = idx_v[f, h, sl]
                        if f < n_band_fields:
                            main = (f_row[f] * v_main) + v
                            tail = (tcb + f * w_tail - v_main) + v
                            off = jnp.where(v < v_main, main, tail)
                        else:
                            off = trb + v
                        idx_v[f, h, sl] = off
            copies = [
                pltpu.async_copy(sp.at[idx_v.at[f, h]], vals_v.at[f, h], sem)
                for f in fields
                for h in range(n_half)
            ]
            for cp in copies:
                cp.wait()
            out = []
            for h in range(n_half):
                for ch in range(n_chunks_h):
                    sl = pl.ds(ch * L, L)
                    acc = vals_v[fields[0], h, sl]
                    for f in fields[1:]:
                        acc = acc + vals_v[f, h, sl]
                    out.append(acc)
            return out

        def store_acc(chunks):
            i = 0
            for h in range(n_half):
                for ch in range(n_chunks_h):
                    acc_v[pl.ds(h * 128 + ch * L, L)] = chunks[i]
                    i += 1

        @pl.when(c == 0)
        def _():
            for si in range(NS):

                @pl.when(s == si)
                def _(si=si):
                    stage_band(0, starts[si], per[si] * 128)
                    stage_tail_row(0, si)
                    if si == NS - 1:
                        stage_tails()

            plsc.subcore_barrier()
            acc0 = offsets_and_gather(
                list(range(0, 8)), {f: f for f in range(8)}
            )
            plsc.subcore_barrier()
            for si in range(NS):

                @pl.when(s == si)
                def _(si=si):
                    stage_band(1, starts[si], per[si] * 128)

            plsc.subcore_barrier()
            acc1 = offsets_and_gather(
                list(range(8, 16)) + [24], {f: f - 8 for f in range(8, 16)}
            )
            store_acc([a + b for a, b in zip(acc0, acc1)])

        @pl.when(c == 1)
        def _():
            for si in range(NS):

                @pl.when(s == si)
                def _(si=si):
                    stage_band(2, starts[si], per[si] * 128)
                    stage_tail_row(V, si)
                    if si == NS - 1:
                        stage_tails()

            plsc.subcore_barrier()
            acc = offsets_and_gather(
                list(range(16, n_band_fields)) + [25],
                {f: f - 16 for f in range(16, n_band_fields)},
            )
            store_acc(acc)

        pltpu.sync_copy(acc_v, out_hbm.at[c, pl.ds(s * rows_pw, rows_pw)])

    partials = sc_kernel(xs_blk, tail_rows, tail_cols, table)

    def tc_body(p_ref, xd_ref, w_ref, o_ref):
        ps = p_ref[0, :] + p_ref[1, :]
        dense = jnp.sum(xd_ref[...] * w_ref[...].T, axis=1, keepdims=True)
        o_ref[...] = ps[:, None] + dense

    return pl.pallas_call(
        tc_body,
        out_shape=jax.ShapeDtypeStruct((B, 1), jnp.float32),
    )(partials, x_dense, W_dense)
